# baseline (speedup 1.0000x reference)
import jax
import jax.numpy as jnp
from jax.experimental import pallas as pl
from jax.experimental.pallas import tpu as pltpu


def _body(x_ref, m_ref, v_ref, b_ref, g_ref, o_ref):
    x = x_ref[...]
    z = jnp.dot(x, v_ref[...], preferred_element_type=jnp.float32) + b_ref[...]
    mf = m_ref[...]
    m0 = mf[:, 0:1]
    m1 = mf[:, 1:2]
    g = g_ref[...]
    mc = m0 + g * (m1 - m0)
    o_ref[...] = x + mc * z


def kernel(in_tensor, mask, W, b):
    M, D = in_tensor.shape
    M2 = M // 2
    BM = 4000
    x2 = in_tensor.reshape(M2, 2 * D)
    m2 = mask.astype(jnp.float32).reshape(M2, 2)
    zeros = jnp.zeros((D, D), dtype=W.dtype)
    v2 = jnp.block([[W, zeros], [zeros, W]]) - jnp.eye(2 * D, dtype=W.dtype)
    b2 = jnp.concatenate([b, b]).reshape(1, 2 * D)
    gsel = jnp.concatenate(
        [jnp.zeros((D,), jnp.float32), jnp.ones((D,), jnp.float32)]
    ).reshape(1, 2 * D)
    out2 = pl.pallas_call(
        _body,
        grid=(M2 // BM,),
        in_specs=[
            pl.BlockSpec((BM, 2 * D), lambda i: (i, 0)),
            pl.BlockSpec((BM, 2), lambda i: (i, 0)),
            pl.BlockSpec((2 * D, 2 * D), lambda i: (0, 0)),
            pl.BlockSpec((1, 2 * D), lambda i: (0, 0)),
            pl.BlockSpec((1, 2 * D), lambda i: (0, 0)),
        ],
        out_specs=pl.BlockSpec((BM, 2 * D), lambda i: (i, 0)),
        out_shape=jax.ShapeDtypeStruct((M2, 2 * D), jnp.float32),
        compiler_params=pltpu.CompilerParams(
            dimension_semantics=("parallel",),
        ),
    )(x2, m2, v2, b2, gsel)
    return out2.reshape(M, D)


# speedup vs baseline: 1.0176x; 1.0176x over previous
import jax
import jax.numpy as jnp
from jax.experimental import pallas as pl
from jax.experimental.pallas import tpu as pltpu

NBUF = 8
CH = 2500


def _body(x_hbm, m_hbm, v_ref, b_ref, g_ref, o_hbm,
          xbuf, mbuf, obuf, xsem, msem, osem):
    i = pl.program_id(0)
    nchunk = pl.num_programs(0)
    slot = jax.lax.rem(i, NBUF)

    def start_in(chunk, s):
        pltpu.make_async_copy(
            x_hbm.at[pl.ds(chunk * CH, CH), :], xbuf.at[s], xsem.at[s]
        ).start()
        pltpu.make_async_copy(
            m_hbm.at[pl.ds(chunk * CH, CH), :], mbuf.at[s], msem.at[s]
        ).start()

    @pl.when(i == 0)
    def _():
        for k in range(NBUF - 1):
            start_in(k, k)

    @pl.when(i + NBUF - 1 < nchunk)
    def _():
        start_in(i + NBUF - 1, jax.lax.rem(i + NBUF - 1, NBUF))

    pltpu.make_async_copy(
        x_hbm.at[pl.ds(i * CH, CH), :], xbuf.at[slot], xsem.at[slot]
    ).wait()
    pltpu.make_async_copy(
        m_hbm.at[pl.ds(i * CH, CH), :], mbuf.at[slot], msem.at[slot]
    ).wait()

    x = xbuf[slot]
    z = jnp.dot(x, v_ref[...], preferred_element_type=jnp.float32) + b_ref[...]
    mf = mbuf[slot]
    mc = mf[:, 0:1] + g_ref[...] * (mf[:, 1:2] - mf[:, 0:1])

    @pl.when(i >= NBUF)
    def _():
        pltpu.make_async_copy(
            obuf.at[slot], o_hbm.at[pl.ds((i - NBUF) * CH, CH), :], osem.at[slot]
        ).wait()

    obuf[slot] = x + mc * z
    pltpu.make_async_copy(
        obuf.at[slot], o_hbm.at[pl.ds(i * CH, CH), :], osem.at[slot]
    ).start()

    @pl.when(i == nchunk - 1)
    def _():
        for k in range(NBUF):
            pltpu.make_async_copy(
                obuf.at[k],
                o_hbm.at[pl.ds((nchunk - NBUF + k) * CH, CH), :],
                osem.at[k],
            ).wait()


def kernel(in_tensor, mask, W, b):
    M, D = in_tensor.shape
    M2 = M // 2
    x2 = in_tensor.reshape(M2, 2 * D)
    m2 = mask.astype(jnp.float32).reshape(M2, 2)
    zeros = jnp.zeros((D, D), dtype=W.dtype)
    v2 = jnp.block([[W, zeros], [zeros, W]]) - jnp.eye(2 * D, dtype=W.dtype)
    b2 = jnp.concatenate([b, b]).reshape(1, 2 * D)
    gsel = jnp.concatenate(
        [jnp.zeros((D,), jnp.float32), jnp.ones((D,), jnp.float32)]
    ).reshape(1, 2 * D)
    out2 = pl.pallas_call(
        _body,
        grid=(M2 // CH,),
        in_specs=[
            pl.BlockSpec(memory_space=pl.ANY),
            pl.BlockSpec(memory_space=pl.ANY),
            pl.BlockSpec(memory_space=pltpu.VMEM),
            pl.BlockSpec(memory_space=pltpu.VMEM),
            pl.BlockSpec(memory_space=pltpu.VMEM),
        ],
        out_specs=pl.BlockSpec(memory_space=pl.ANY),
        out_shape=jax.ShapeDtypeStruct((M2, 2 * D), jnp.float32),
        scratch_shapes=[
            pltpu.VMEM((NBUF, CH, 2 * D), jnp.float32),
            pltpu.VMEM((NBUF, CH, 2), jnp.float32),
            pltpu.VMEM((NBUF, CH, 2 * D), jnp.float32),
            pltpu.SemaphoreType.DMA((NBUF,)),
            pltpu.SemaphoreType.DMA((NBUF,)),
            pltpu.SemaphoreType.DMA((NBUF,)),
        ],
        compiler_params=pltpu.CompilerParams(
            dimension_semantics=("arbitrary",),
        ),
    )(x2, m2, v2, b2, gsel)
    return out2.reshape(M, D)


# speedup vs baseline: 1.0779x; 1.0593x over previous
import jax
import jax.numpy as jnp
from jax.experimental import pallas as pl
from jax.experimental.pallas import tpu as pltpu

K = 10
BM = 2000


def _body(*refs):
    x_refs = refs[:K]
    m_refs = refs[K:2 * K]
    w_ref, b_ref, o_hbm, obuf = refs[2 * K:2 * K + 4]
    osem = refs[2 * K + 4]

    i = pl.program_id(0)
    nsteps = pl.num_programs(0)
    shard_rows = nsteps * BM

    @pl.when(i > 0)
    def _():
        for k in range(K):
            pltpu.make_async_copy(
                obuf.at[k],
                o_hbm.at[pl.ds(k * shard_rows + (i - 1) * BM, BM), :],
                osem.at[k],
            ).wait()

    w = w_ref[...]
    b = b_ref[...]
    for k in range(K):
        x = x_refs[k][0]
        y = jnp.dot(x, w, preferred_element_type=jnp.float32) + b
        m = m_refs[k][0, 0]
        obuf[k] = jnp.where(m, y, x)
        pltpu.make_async_copy(
            obuf.at[k],
            o_hbm.at[pl.ds(k * shard_rows + i * BM, BM), :],
            osem.at[k],
        ).start()

    @pl.when(i == nsteps - 1)
    def _():
        for k in range(K):
            pltpu.make_async_copy(
                obuf.at[k],
                o_hbm.at[pl.ds(k * shard_rows + i * BM, BM), :],
                osem.at[k],
            ).wait()


def kernel(in_tensor, mask, W, b):
    M, D = in_tensor.shape
    shard = M // K
    nsteps = shard // BM
    x3 = in_tensor.reshape(K, shard, D)
    m4 = mask.reshape(K, nsteps, BM, 1)
    b2 = b.reshape(1, D)

    x_specs = [
        pl.BlockSpec((1, BM, D), lambda i, j=j: (j, i, 0)) for j in range(K)
    ]
    m_specs = [
        pl.BlockSpec((1, 1, BM, 1), lambda i, j=j: (j, i, 0, 0)) for j in range(K)
    ]
    return pl.pallas_call(
        _body,
        grid=(nsteps,),
        in_specs=x_specs + m_specs + [
            pl.BlockSpec(memory_space=pltpu.VMEM),
            pl.BlockSpec(memory_space=pltpu.VMEM),
        ],
        out_specs=pl.BlockSpec(memory_space=pl.ANY),
        out_shape=jax.ShapeDtypeStruct((M, D), jnp.float32),
        scratch_shapes=[
            pltpu.VMEM((K, BM, D), jnp.float32),
            pltpu.SemaphoreType.DMA((K,)),
        ],
        compiler_params=pltpu.CompilerParams(
            dimension_semantics=("arbitrary",),
        ),
    )(*([x3] * K), *([m4] * K), W, b2)
